# baseline (speedup 1.0000x reference)
import functools

import jax
import jax.numpy as jnp
from jax import lax
from jax.experimental import pallas as pl
from jax.experimental.pallas import tpu as pltpu
from jax.experimental.pallas import tpu_sc as plsc

D = 128
NC = 2
NS = 16
NW = NC * NS
CHUNK = 80
NB = 4
HGS = 5
NPAD = 10240


def _tc_gru_xw(x, w0, w_ih, w_hh, b_ih2, b_hh2):
    def body(x_ref, w0_ref, wih_ref, whh_ref, bih_ref, bhh_ref, out_ref):
        w = w0_ref[...]
        gi = lax.dot_general(w, wih_ref[...], (((1,), (1,)), ((), ())),
                             preferred_element_type=jnp.float32) + bih_ref[...]
        gh = lax.dot_general(w, whh_ref[...], (((1,), (1,)), ((), ())),
                             preferred_element_type=jnp.float32) + bhh_ref[...]
        r = jax.nn.sigmoid(gi[:, :D] + gh[:, :D])
        z = jax.nn.sigmoid(gi[:, D:2 * D] + gh[:, D:2 * D])
        n = jnp.tanh(gi[:, 2 * D:] + r * gh[:, 2 * D:])
        wt = (1.0 - z) * n + z * w
        out_ref[...] = jnp.dot(x_ref[...], wt,
                               preferred_element_type=jnp.float32)

    return pl.pallas_call(
        body,
        out_shape=jax.ShapeDtypeStruct((x.shape[0], D), jnp.float32),
    )(x, w0, w_ih, w_hh, b_ih2, b_hh2)


def _sc_degree(dst_h):
    e = dst_h.shape[0]
    ept = e // NW
    ngrp = ept // (HGS * CHUNK)
    spt = NPAD // NS
    mesh = plsc.VectorSubcoreMesh(core_axis_name="c", subcore_axis_name="s")

    @functools.partial(
        pl.kernel,
        out_type=jax.ShapeDtypeStruct((NC, 1, NPAD), jnp.float32),
        mesh=mesh,
        scratch_types=[
            pltpu.VMEM((HGS, CHUNK), jnp.int32),
            pltpu.VMEM((CHUNK,), jnp.float32),
            pltpu.VMEM((spt,), jnp.float32),
            pltpu.VMEM_SHARED((NPAD,), jnp.float32),
            pltpu.SemaphoreType.DMA,
        ],
    )
    def k(dst_hbm, out_hbm, idx_v, ones_v, buf_v, acc_sh, sem):
        c = lax.axis_index("c")
        s = lax.axis_index("s")
        wid = s * NC + c
        base = wid * ept

        def fill_zero(i, _):
            buf_v[pl.ds(i * 16, 16)] = jnp.zeros((16,), jnp.float32)
            return 0
        lax.fori_loop(0, spt // 16, fill_zero, 0)

        def fill_one(i, _):
            ones_v[pl.ds(i * 16, 16)] = jnp.ones((16,), jnp.float32)
            return 0
        lax.fori_loop(0, CHUNK // 16, fill_one, 0)

        pltpu.sync_copy(buf_v, acc_sh.at[pl.ds(s * spt, spt)])
        plsc.subcore_barrier()

        def grp(g, _):
            off = base + g * (HGS * CHUNK)
            for t in range(HGS):
                pltpu.sync_copy(dst_hbm.at[pl.ds(off + t * CHUNK, CHUNK)],
                                idx_v.at[t])
            for t in range(HGS):
                pltpu.async_copy(ones_v, acc_sh.at[idx_v.at[t]], sem, add=True)
            for t in range(HGS):
                pltpu.make_async_copy(ones_v, acc_sh.at[idx_v.at[t]], sem).wait()
            return 0
        lax.fori_loop(0, ngrp, grp, 0)
        plsc.subcore_barrier()

        pltpu.sync_copy(acc_sh.at[pl.ds(s * spt, spt)], buf_v)
        pltpu.sync_copy(buf_v, out_hbm.at[c, 0, pl.ds(s * spt, spt)])

    return k(dst_h)


def _sc_scatter(src_h, dst_h, y, zeros_h):
    e = src_h.shape[0]
    ept = e // NW
    nch = ept // CHUNK
    ngrp = nch // NB
    tail = nch - ngrp * NB
    npt = NPAD // NS
    mesh = plsc.VectorSubcoreMesh(core_axis_name="c", subcore_axis_name="s")

    @functools.partial(
        pl.kernel,
        out_type=jax.ShapeDtypeStruct((NC, NPAD, D), jnp.float32),
        mesh=mesh,
        scratch_types=[
            pltpu.VMEM((3, NB, CHUNK), jnp.int32),
            pltpu.VMEM((3, NB, CHUNK), jnp.int32),
            pltpu.VMEM((NB, CHUNK, D), jnp.float32),
            pltpu.VMEM_SHARED((NPAD, D), jnp.float32),
            [pltpu.SemaphoreType.DMA] * NB,
            [pltpu.SemaphoreType.DMA] * NB,
            [pltpu.SemaphoreType.DMA] * 2,
        ],
    )
    def k(src_hbm, dst_hbm, y_hbm, z_hbm, out_hbm,
          si_v, di_v, rows_v, acc_sh, gsems, ssems, stsems):
        c = lax.axis_index("c")
        s = lax.axis_index("s")
        wid = s * NC + c
        base = wid * ept
        r0 = s * npt

        def issue_stage(m, sem):
            b = lax.rem(m, 3)
            off = base + m * (NB * CHUNK)
            for t in range(NB):
                pltpu.async_copy(
                    src_hbm.at[pl.ds(off + t * CHUNK, CHUNK)],
                    si_v.at[b, t], sem)
            for t in range(NB):
                pltpu.async_copy(
                    dst_hbm.at[pl.ds(off + t * CHUNK, CHUNK)],
                    di_v.at[b, t], sem)

        def wait_stage(bg, sem):
            for t in range(NB):
                pltpu.make_async_copy(src_hbm.at[pl.ds(0, CHUNK)],
                                      si_v.at[bg, t], sem).wait()
            for t in range(NB):
                pltpu.make_async_copy(dst_hbm.at[pl.ds(0, CHUNK)],
                                      di_v.at[bg, t], sem).wait()

        def wait_gather(b):
            pltpu.make_async_copy(
                y_hbm.at[si_v.at[0, 0]], rows_v.at[b], gsems[b]).wait()

        def wait_scatter(b):
            pltpu.make_async_copy(
                rows_v.at[b], acc_sh.at[di_v.at[0, 0]], ssems[b]).wait()

        pltpu.sync_copy(z_hbm, acc_sh.at[pl.ds(r0, npt), :])
        for t in range(NB):
            pltpu.sync_copy(src_hbm.at[pl.ds(base + t * CHUNK, CHUNK)],
                            si_v.at[0, t])
            pltpu.sync_copy(dst_hbm.at[pl.ds(base + t * CHUNK, CHUNK)],
                            di_v.at[0, t])
        issue_stage(1, stsems[1])
        issue_stage(2, stsems[0])
        pltpu.async_copy(y_hbm.at[si_v.at[0, 0]], rows_v.at[0], gsems[0])
        pltpu.async_copy(y_hbm.at[si_v.at[0, 1]], rows_v.at[1], gsems[1])
        plsc.subcore_barrier()

        def grp(g, _):
            bg = lax.rem(g, 3)
            bg1 = lax.rem(g + 1, 3)
            even = lax.rem(g, 2) == 0

            @pl.when(jnp.logical_and(g > 0, even))
            def _ws0():
                wait_stage(bg, stsems[0])

            @pl.when(jnp.logical_and(g > 0, jnp.logical_not(even)))
            def _ws1():
                wait_stage(bg, stsems[1])

            for t in range(NB):
                b = t % NB
                jb = (t + 2) % NB
                if t < 2:
                    @pl.when(g > 0)
                    def _wsct():
                        wait_scatter(jb)
                else:
                    wait_scatter(jb)
                if t == 2:
                    @pl.when(jnp.logical_and(g + 2 < ngrp, even))
                    def _st0():
                        issue_stage(g + 2, stsems[0])

                    @pl.when(jnp.logical_and(g + 2 < ngrp,
                                             jnp.logical_not(even)))
                    def _st1():
                        issue_stage(g + 2, stsems[1])
                if t < NB - 2:
                    nxt = si_v.at[bg, t + 2]
                else:
                    nxt = si_v.at[bg1, t + 2 - NB]
                pltpu.async_copy(y_hbm.at[nxt], rows_v.at[jb], gsems[jb])
                wait_gather(b)
                pltpu.async_copy(rows_v.at[b], acc_sh.at[di_v.at[bg, t]],
                                 ssems[b], add=True)
            return 0
        lax.fori_loop(0, ngrp, grp, 0)
        wait_scatter(NB - 2)
        wait_scatter(NB - 1)
        wait_gather(0)
        wait_gather(1)
        for u in range(tail):
            off = base + (ngrp * NB + u) * CHUNK
            pltpu.sync_copy(src_hbm.at[pl.ds(off, CHUNK)], si_v.at[0, 0])
            pltpu.sync_copy(dst_hbm.at[pl.ds(off, CHUNK)], di_v.at[0, 0])
            pltpu.async_copy(y_hbm.at[si_v.at[0, 0]], rows_v.at[0], gsems[0])
            wait_gather(0)
            pltpu.sync_copy(rows_v.at[0], acc_sh.at[di_v.at[0, 0]], add=True)
        plsc.subcore_barrier()

        pltpu.sync_copy(acc_sh.at[pl.ds(r0, npt), :],
                        out_hbm.at[c, pl.ds(r0, npt), :])

    return k(src_h, dst_h, y, zeros_h)


def _tc_scale(xw, degp):
    n = xw.shape[0]

    def body(xw_ref, degp_ref, y_ref):
        deg = degp_ref[0, 0] + degp_ref[1, 0] + 1.0
        dinv = lax.rsqrt(deg)
        y_ref[:n] = xw_ref[...] * dinv[:n][:, None]
        y_ref[n:] = jnp.zeros((NPAD - n, D), jnp.float32)

    return pl.pallas_call(
        body,
        out_shape=jax.ShapeDtypeStruct((NPAD, D), jnp.float32),
    )(xw, degp)


def _tc_final(partials, y, degp, n):
    def body(p_ref, y_ref, degp_ref, out_ref):
        deg = degp_ref[0, 0] + degp_ref[1, 0] + 1.0
        dinv = lax.rsqrt(deg)
        acc = p_ref[0][:n] + p_ref[1][:n] + y_ref[:n]
        out_ref[...] = acc * dinv[:n][:, None]

    return pl.pallas_call(
        body,
        out_shape=jax.ShapeDtypeStruct((n, D), jnp.float32),
    )(partials, y, degp)


def kernel(x, edge_index, initial_weight, w_ih, w_hh, b_ih, b_hh):
    n = x.shape[0]
    e = edge_index.shape[1]
    assert e % (NW * HGS * CHUNK) == 0 and n < NPAD

    b_ih2 = b_ih.reshape(1, 3 * D)
    b_hh2 = b_hh.reshape(1, 3 * D)
    zeros_h = jnp.zeros((NPAD // NS, D), jnp.float32)

    src_h = edge_index[0]
    dst_h = edge_index[1]
    degp = _sc_degree(dst_h)
    xw = _tc_gru_xw(x, initial_weight, w_ih, w_hh, b_ih2, b_hh2)
    y = _tc_scale(xw, degp)
    partials = _sc_scatter(src_h, dst_h, y, zeros_h)
    return _tc_final(partials, y, degp, n)


# speedup vs baseline: 1.2470x; 1.2470x over previous
import functools

import jax
import jax.numpy as jnp
from jax import lax
from jax.experimental import pallas as pl
from jax.experimental.pallas import tpu as pltpu
from jax.experimental.pallas import tpu_sc as plsc

D = 128
NC = 2
NS = 16
NW = NC * NS
CHUNK = 80
NB = 4
HGS = 5
NPAD = 10240


def _tc_gru_xw(x, w0, w_ih, w_hh, b_ih2, b_hh2):
    def body(x_ref, w0_ref, wih_ref, whh_ref, bih_ref, bhh_ref, out_ref):
        w = w0_ref[...]
        gi = lax.dot_general(w, wih_ref[...], (((1,), (1,)), ((), ())),
                             preferred_element_type=jnp.float32) + bih_ref[...]
        gh = lax.dot_general(w, whh_ref[...], (((1,), (1,)), ((), ())),
                             preferred_element_type=jnp.float32) + bhh_ref[...]
        r = jax.nn.sigmoid(gi[:, :D] + gh[:, :D])
        z = jax.nn.sigmoid(gi[:, D:2 * D] + gh[:, D:2 * D])
        n = jnp.tanh(gi[:, 2 * D:] + r * gh[:, 2 * D:])
        wt = (1.0 - z) * n + z * w
        out_ref[...] = jnp.dot(x_ref[...], wt,
                               preferred_element_type=jnp.float32)

    return pl.pallas_call(
        body,
        out_shape=jax.ShapeDtypeStruct((x.shape[0], D), jnp.float32),
    )(x, w0, w_ih, w_hh, b_ih2, b_hh2)


def _sc_degree(dst_h):
    e = dst_h.shape[0]
    ept = e // NW
    ngrp = ept // (HGS * CHUNK)
    spt = NPAD // NS
    mesh = plsc.VectorSubcoreMesh(core_axis_name="c", subcore_axis_name="s")

    @functools.partial(
        pl.kernel,
        out_type=jax.ShapeDtypeStruct((NC, 1, NPAD), jnp.float32),
        mesh=mesh,
        scratch_types=[
            pltpu.VMEM((2, HGS, CHUNK), jnp.int32),
            pltpu.VMEM((CHUNK,), jnp.float32),
            pltpu.VMEM((spt,), jnp.float32),
            pltpu.VMEM_SHARED((NPAD,), jnp.float32),
            pltpu.SemaphoreType.DMA,
            [pltpu.SemaphoreType.DMA] * 2,
        ],
    )
    def k(dst_hbm, out_hbm, idx_v, ones_v, buf_v, acc_sh, sem, stsems):
        c = lax.axis_index("c")
        s = lax.axis_index("s")
        wid = s * NC + c
        base = wid * ept

        def fill_zero(i, _):
            buf_v[pl.ds(i * 16, 16)] = jnp.zeros((16,), jnp.float32)
            return 0
        lax.fori_loop(0, spt // 16, fill_zero, 0)

        def fill_one(i, _):
            ones_v[pl.ds(i * 16, 16)] = jnp.ones((16,), jnp.float32)
            return 0
        lax.fori_loop(0, CHUNK // 16, fill_one, 0)

        def issue_stage(m, buf, sem2):
            off = base + m * (HGS * CHUNK)
            for t in range(HGS):
                pltpu.async_copy(dst_hbm.at[pl.ds(off + t * CHUNK, CHUNK)],
                                 idx_v.at[buf, t], sem2)

        def wait_stage(buf, sem2):
            for t in range(HGS):
                pltpu.make_async_copy(dst_hbm.at[pl.ds(0, CHUNK)],
                                      idx_v.at[buf, t], sem2).wait()

        pltpu.sync_copy(buf_v, acc_sh.at[pl.ds(s * spt, spt)])
        for t in range(HGS):
            pltpu.sync_copy(dst_hbm.at[pl.ds(base + t * CHUNK, CHUNK)],
                            idx_v.at[0, t])
        plsc.subcore_barrier()

        def grp(g, _):
            p = lax.rem(g, 2)
            even = p == 0

            @pl.when(jnp.logical_and(g > 0, even))
            def _w0():
                wait_stage(p, stsems[0])

            @pl.when(jnp.logical_and(g > 0, jnp.logical_not(even)))
            def _w1():
                wait_stage(p, stsems[1])

            @pl.when(jnp.logical_and(g + 1 < ngrp, even))
            def _s1():
                issue_stage(g + 1, 1 - p, stsems[1])

            @pl.when(jnp.logical_and(g + 1 < ngrp, jnp.logical_not(even)))
            def _s0():
                issue_stage(g + 1, 1 - p, stsems[0])

            for t in range(HGS):
                pltpu.async_copy(ones_v, acc_sh.at[idx_v.at[p, t]], sem,
                                 add=True)
            for t in range(HGS):
                pltpu.make_async_copy(ones_v, acc_sh.at[idx_v.at[p, t]],
                                      sem).wait()
            return 0
        lax.fori_loop(0, ngrp, grp, 0)
        plsc.subcore_barrier()

        pltpu.sync_copy(acc_sh.at[pl.ds(s * spt, spt)], buf_v)
        pltpu.sync_copy(buf_v, out_hbm.at[c, 0, pl.ds(s * spt, spt)])

    return k(dst_h)


def _sc_scatter(src_h, dst_h, y, zeros_h):
    e = src_h.shape[0]
    ept = e // NW
    nch = ept // CHUNK
    ngrp = nch // NB
    tail = nch - ngrp * NB
    npt = NPAD // NS
    mesh = plsc.VectorSubcoreMesh(core_axis_name="c", subcore_axis_name="s")

    @functools.partial(
        pl.kernel,
        out_type=jax.ShapeDtypeStruct((NC, NPAD, D), jnp.float32),
        mesh=mesh,
        scratch_types=[
            pltpu.VMEM((3, NB, CHUNK), jnp.int32),
            pltpu.VMEM((3, NB, CHUNK), jnp.int32),
            pltpu.VMEM((NB, CHUNK, D), jnp.float32),
            pltpu.VMEM_SHARED((NPAD, D), jnp.float32),
            [pltpu.SemaphoreType.DMA] * NB,
            [pltpu.SemaphoreType.DMA] * NB,
            [pltpu.SemaphoreType.DMA] * 2,
        ],
    )
    def k(src_hbm, dst_hbm, y_hbm, z_hbm, out_hbm,
          si_v, di_v, rows_v, acc_sh, gsems, ssems, stsems):
        c = lax.axis_index("c")
        s = lax.axis_index("s")
        wid = s * NC + c
        base = wid * ept
        r0 = s * npt

        def issue_stage(m, sem):
            b = lax.rem(m, 3)
            off = base + m * (NB * CHUNK)
            for t in range(NB):
                pltpu.async_copy(
                    src_hbm.at[pl.ds(off + t * CHUNK, CHUNK)],
                    si_v.at[b, t], sem)
            for t in range(NB):
                pltpu.async_copy(
                    dst_hbm.at[pl.ds(off + t * CHUNK, CHUNK)],
                    di_v.at[b, t], sem)

        def wait_stage(bg, sem):
            for t in range(NB):
                pltpu.make_async_copy(src_hbm.at[pl.ds(0, CHUNK)],
                                      si_v.at[bg, t], sem).wait()
            for t in range(NB):
                pltpu.make_async_copy(dst_hbm.at[pl.ds(0, CHUNK)],
                                      di_v.at[bg, t], sem).wait()

        def wait_gather(b):
            pltpu.make_async_copy(
                y_hbm.at[si_v.at[0, 0]], rows_v.at[b], gsems[b]).wait()

        def wait_scatter(b):
            pltpu.make_async_copy(
                rows_v.at[b], acc_sh.at[di_v.at[0, 0]], ssems[b]).wait()

        pltpu.sync_copy(z_hbm, acc_sh.at[pl.ds(r0, npt), :])
        for t in range(NB):
            pltpu.sync_copy(src_hbm.at[pl.ds(base + t * CHUNK, CHUNK)],
                            si_v.at[0, t])
            pltpu.sync_copy(dst_hbm.at[pl.ds(base + t * CHUNK, CHUNK)],
                            di_v.at[0, t])
        issue_stage(1, stsems[1])
        issue_stage(2, stsems[0])
        pltpu.async_copy(y_hbm.at[si_v.at[0, 0]], rows_v.at[0], gsems[0])
        pltpu.async_copy(y_hbm.at[si_v.at[0, 1]], rows_v.at[1], gsems[1])
        plsc.subcore_barrier()

        def grp(g, _):
            bg = lax.rem(g, 3)
            bg1 = lax.rem(g + 1, 3)
            even = lax.rem(g, 2) == 0

            @pl.when(jnp.logical_and(g > 0, even))
            def _ws0():
                wait_stage(bg, stsems[0])

            @pl.when(jnp.logical_and(g > 0, jnp.logical_not(even)))
            def _ws1():
                wait_stage(bg, stsems[1])

            for t in range(NB):
                b = t % NB
                jb = (t + 2) % NB
                if t < 2:
                    @pl.when(g > 0)
                    def _wsct():
                        wait_scatter(jb)
                else:
                    wait_scatter(jb)
                if t == 2:
                    @pl.when(jnp.logical_and(g + 2 < ngrp, even))
                    def _st0():
                        issue_stage(g + 2, stsems[0])

                    @pl.when(jnp.logical_and(g + 2 < ngrp,
                                             jnp.logical_not(even)))
                    def _st1():
                        issue_stage(g + 2, stsems[1])
                if t < NB - 2:
                    nxt = si_v.at[bg, t + 2]
                else:
                    nxt = si_v.at[bg1, t + 2 - NB]
                pltpu.async_copy(y_hbm.at[nxt], rows_v.at[jb], gsems[jb])
                wait_gather(b)
                pltpu.async_copy(rows_v.at[b], acc_sh.at[di_v.at[bg, t]],
                                 ssems[b], add=True)
            return 0
        lax.fori_loop(0, ngrp, grp, 0)
        wait_scatter(NB - 2)
        wait_scatter(NB - 1)
        wait_gather(0)
        wait_gather(1)
        for u in range(tail):
            off = base + (ngrp * NB + u) * CHUNK
            pltpu.sync_copy(src_hbm.at[pl.ds(off, CHUNK)], si_v.at[0, 0])
            pltpu.sync_copy(dst_hbm.at[pl.ds(off, CHUNK)], di_v.at[0, 0])
            pltpu.async_copy(y_hbm.at[si_v.at[0, 0]], rows_v.at[0], gsems[0])
            wait_gather(0)
            pltpu.sync_copy(rows_v.at[0], acc_sh.at[di_v.at[0, 0]], add=True)
        plsc.subcore_barrier()

        pltpu.sync_copy(acc_sh.at[pl.ds(r0, npt), :],
                        out_hbm.at[c, pl.ds(r0, npt), :])

    return k(src_h, dst_h, y, zeros_h)


def _tc_scale(xw, degp):
    n = xw.shape[0]

    def body(xw_ref, degp_ref, y_ref):
        deg = degp_ref[0, 0] + degp_ref[1, 0] + 1.0
        dinv = lax.rsqrt(deg)
        y_ref[:n] = xw_ref[...] * dinv[:n][:, None]
        y_ref[n:] = jnp.zeros((NPAD - n, D), jnp.float32)

    return pl.pallas_call(
        body,
        out_shape=jax.ShapeDtypeStruct((NPAD, D), jnp.float32),
    )(xw, degp)


def _tc_final(partials, y, degp, n):
    def body(p_ref, y_ref, degp_ref, out_ref):
        deg = degp_ref[0, 0] + degp_ref[1, 0] + 1.0
        dinv = lax.rsqrt(deg)
        acc = p_ref[0][:n] + p_ref[1][:n] + y_ref[:n]
        out_ref[...] = acc * dinv[:n][:, None]

    return pl.pallas_call(
        body,
        out_shape=jax.ShapeDtypeStruct((n, D), jnp.float32),
    )(partials, y, degp)


def kernel(x, edge_index, initial_weight, w_ih, w_hh, b_ih, b_hh):
    n = x.shape[0]
    e = edge_index.shape[1]
    assert e % (NW * HGS * CHUNK) == 0 and n < NPAD

    b_ih2 = b_ih.reshape(1, 3 * D)
    b_hh2 = b_hh.reshape(1, 3 * D)
    zeros_h = jnp.zeros((NPAD // NS, D), jnp.float32)

    src_h = edge_index[0]
    dst_h = edge_index[1]
    degp = _sc_degree(dst_h)
    xw = _tc_gru_xw(x, initial_weight, w_ih, w_hh, b_ih2, b_hh2)
    y = _tc_scale(xw, degp)
    partials = _sc_scatter(src_h, dst_h, y, zeros_h)
    return _tc_final(partials, y, degp, n)


# speedup vs baseline: 1.3352x; 1.0707x over previous
import functools

import jax
import jax.numpy as jnp
from jax import lax
from jax.experimental import pallas as pl
from jax.experimental.pallas import tpu as pltpu
from jax.experimental.pallas import tpu_sc as plsc

D = 128
NC = 2
NS = 16
NW = NC * NS
CHUNK = 80
NB = 4
HGS = 5
NPAD = 10240


def _tc_gru_xw(x, w0, w_ih, w_hh, b_ih2, b_hh2):
    def body(x_ref, w0_ref, wih_ref, whh_ref, bih_ref, bhh_ref, out_ref):
        w = w0_ref[...]
        gi = lax.dot_general(w, wih_ref[...], (((1,), (1,)), ((), ())),
                             preferred_element_type=jnp.float32) + bih_ref[...]
        gh = lax.dot_general(w, whh_ref[...], (((1,), (1,)), ((), ())),
                             preferred_element_type=jnp.float32) + bhh_ref[...]
        r = jax.nn.sigmoid(gi[:, :D] + gh[:, :D])
        z = jax.nn.sigmoid(gi[:, D:2 * D] + gh[:, D:2 * D])
        n = jnp.tanh(gi[:, 2 * D:] + r * gh[:, 2 * D:])
        wt = (1.0 - z) * n + z * w
        out_ref[...] = jnp.dot(x_ref[...], wt,
                               preferred_element_type=jnp.float32)

    return pl.pallas_call(
        body,
        out_shape=jax.ShapeDtypeStruct((x.shape[0], D), jnp.float32),
    )(x, w0, w_ih, w_hh, b_ih2, b_hh2)


def _tc_split(edges):
    e = edges.shape[1]

    def body(e_ref, s_ref, d_ref):
        s_ref[...] = e_ref[0]
        d_ref[...] = e_ref[1]

    return pl.pallas_call(
        body,
        out_shape=(
            jax.ShapeDtypeStruct((e,), jnp.int32),
            jax.ShapeDtypeStruct((e,), jnp.int32),
        ),
    )(edges)


def _sc_degree(dst_h):
    e = dst_h.shape[0]
    ept = e // NW
    ngrp = ept // (HGS * CHUNK)
    spt = NPAD // NS
    mesh = plsc.VectorSubcoreMesh(core_axis_name="c", subcore_axis_name="s")

    @functools.partial(
        pl.kernel,
        out_type=jax.ShapeDtypeStruct((NC, 1, NPAD), jnp.float32),
        mesh=mesh,
        scratch_types=[
            pltpu.VMEM((2, HGS, CHUNK), jnp.int32),
            pltpu.VMEM((CHUNK,), jnp.float32),
            pltpu.VMEM((spt,), jnp.float32),
            pltpu.VMEM_SHARED((NPAD,), jnp.float32),
            pltpu.SemaphoreType.DMA,
            [pltpu.SemaphoreType.DMA] * 2,
        ],
    )
    def k(dst_hbm, out_hbm, idx_v, ones_v, buf_v, acc_sh, sem, stsems):
        c = lax.axis_index("c")
        s = lax.axis_index("s")
        wid = s * NC + c
        base = wid * ept

        def fill_zero(i, _):
            buf_v[pl.ds(i * 16, 16)] = jnp.zeros((16,), jnp.float32)
            return 0
        lax.fori_loop(0, spt // 16, fill_zero, 0)

        def fill_one(i, _):
            ones_v[pl.ds(i * 16, 16)] = jnp.ones((16,), jnp.float32)
            return 0
        lax.fori_loop(0, CHUNK // 16, fill_one, 0)

        def issue_stage(m, buf, sem2):
            off = base + m * (HGS * CHUNK)
            for t in range(HGS):
                pltpu.async_copy(dst_hbm.at[pl.ds(off + t * CHUNK, CHUNK)],
                                 idx_v.at[buf, t], sem2)

        def wait_stage(buf, sem2):
            for t in range(HGS):
                pltpu.make_async_copy(dst_hbm.at[pl.ds(0, CHUNK)],
                                      idx_v.at[buf, t], sem2).wait()

        pltpu.sync_copy(buf_v, acc_sh.at[pl.ds(s * spt, spt)])
        for t in range(HGS):
            pltpu.sync_copy(dst_hbm.at[pl.ds(base + t * CHUNK, CHUNK)],
                            idx_v.at[0, t])
        plsc.subcore_barrier()

        def grp(g, _):
            p = lax.rem(g, 2)
            even = p == 0

            @pl.when(jnp.logical_and(g > 0, even))
            def _w0():
                wait_stage(p, stsems[0])

            @pl.when(jnp.logical_and(g > 0, jnp.logical_not(even)))
            def _w1():
                wait_stage(p, stsems[1])

            @pl.when(jnp.logical_and(g + 1 < ngrp, even))
            def _s1():
                issue_stage(g + 1, 1 - p, stsems[1])

            @pl.when(jnp.logical_and(g + 1 < ngrp, jnp.logical_not(even)))
            def _s0():
                issue_stage(g + 1, 1 - p, stsems[0])

            for t in range(HGS):
                pltpu.async_copy(ones_v, acc_sh.at[idx_v.at[p, t]], sem,
                                 add=True)
            for t in range(HGS):
                pltpu.make_async_copy(ones_v, acc_sh.at[idx_v.at[p, t]],
                                      sem).wait()
            return 0
        lax.fori_loop(0, ngrp, grp, 0)
        plsc.subcore_barrier()

        pltpu.sync_copy(acc_sh.at[pl.ds(s * spt, spt)], buf_v)
        pltpu.sync_copy(buf_v, out_hbm.at[c, 0, pl.ds(s * spt, spt)])

    return k(dst_h)


def _sc_scatter(src_h, dst_h, y, zeros_h):
    e = src_h.shape[0]
    ept = e // NW
    nch = ept // CHUNK
    ngrp = nch // NB
    tail = nch - ngrp * NB
    npt = NPAD // NS
    mesh = plsc.VectorSubcoreMesh(core_axis_name="c", subcore_axis_name="s")

    @functools.partial(
        pl.kernel,
        out_type=jax.ShapeDtypeStruct((NC, NPAD, D), jnp.float32),
        mesh=mesh,
        scratch_types=[
            pltpu.VMEM((3, NB, CHUNK), jnp.int32),
            pltpu.VMEM((3, NB, CHUNK), jnp.int32),
            pltpu.VMEM((NB, CHUNK, D), jnp.float32),
            pltpu.VMEM_SHARED((NPAD, D), jnp.float32),
            [pltpu.SemaphoreType.DMA] * NB,
            [pltpu.SemaphoreType.DMA] * NB,
            [pltpu.SemaphoreType.DMA] * 2,
        ],
    )
    def k(src_hbm, dst_hbm, y_hbm, z_hbm, out_hbm,
          si_v, di_v, rows_v, acc_sh, gsems, ssems, stsems):
        c = lax.axis_index("c")
        s = lax.axis_index("s")
        wid = s * NC + c
        base = wid * ept
        r0 = s * npt

        def issue_stage(m, sem):
            b = lax.rem(m, 3)
            off = base + m * (NB * CHUNK)
            for t in range(NB):
                pltpu.async_copy(
                    src_hbm.at[pl.ds(off + t * CHUNK, CHUNK)],
                    si_v.at[b, t], sem)
            for t in range(NB):
                pltpu.async_copy(
                    dst_hbm.at[pl.ds(off + t * CHUNK, CHUNK)],
                    di_v.at[b, t], sem)

        def wait_stage(bg, sem):
            for t in range(NB):
                pltpu.make_async_copy(src_hbm.at[pl.ds(0, CHUNK)],
                                      si_v.at[bg, t], sem).wait()
            for t in range(NB):
                pltpu.make_async_copy(dst_hbm.at[pl.ds(0, CHUNK)],
                                      di_v.at[bg, t], sem).wait()

        def wait_gather(b):
            pltpu.make_async_copy(
                y_hbm.at[si_v.at[0, 0]], rows_v.at[b], gsems[b]).wait()

        def wait_scatter(b):
            pltpu.make_async_copy(
                rows_v.at[b], acc_sh.at[di_v.at[0, 0]], ssems[b]).wait()

        pltpu.sync_copy(z_hbm, acc_sh.at[pl.ds(r0, npt), :])
        for t in range(NB):
            pltpu.sync_copy(src_hbm.at[pl.ds(base + t * CHUNK, CHUNK)],
                            si_v.at[0, t])
            pltpu.sync_copy(dst_hbm.at[pl.ds(base + t * CHUNK, CHUNK)],
                            di_v.at[0, t])
        issue_stage(1, stsems[1])
        issue_stage(2, stsems[0])
        pltpu.async_copy(y_hbm.at[si_v.at[0, 0]], rows_v.at[0], gsems[0])
        pltpu.async_copy(y_hbm.at[si_v.at[0, 1]], rows_v.at[1], gsems[1])
        plsc.subcore_barrier()

        def grp(g, _):
            bg = lax.rem(g, 3)
            bg1 = lax.rem(g + 1, 3)
            even = lax.rem(g, 2) == 0

            @pl.when(jnp.logical_and(g > 0, even))
            def _ws0():
                wait_stage(bg, stsems[0])

            @pl.when(jnp.logical_and(g > 0, jnp.logical_not(even)))
            def _ws1():
                wait_stage(bg, stsems[1])

            for t in range(NB):
                b = t % NB
                jb = (t + 2) % NB
                if t < 2:
                    @pl.when(g > 0)
                    def _wsct():
                        wait_scatter(jb)
                else:
                    wait_scatter(jb)
                if t == 2:
                    @pl.when(jnp.logical_and(g + 2 < ngrp, even))
                    def _st0():
                        issue_stage(g + 2, stsems[0])

                    @pl.when(jnp.logical_and(g + 2 < ngrp,
                                             jnp.logical_not(even)))
                    def _st1():
                        issue_stage(g + 2, stsems[1])
                if t < NB - 2:
                    nxt = si_v.at[bg, t + 2]
                else:
                    nxt = si_v.at[bg1, t + 2 - NB]
                pltpu.async_copy(y_hbm.at[nxt], rows_v.at[jb], gsems[jb])
                wait_gather(b)
                pltpu.async_copy(rows_v.at[b], acc_sh.at[di_v.at[bg, t]],
                                 ssems[b], add=True)
            return 0
        lax.fori_loop(0, ngrp, grp, 0)
        wait_scatter(NB - 2)
        wait_scatter(NB - 1)
        wait_gather(0)
        wait_gather(1)
        for u in range(tail):
            off = base + (ngrp * NB + u) * CHUNK
            pltpu.sync_copy(src_hbm.at[pl.ds(off, CHUNK)], si_v.at[0, 0])
            pltpu.sync_copy(dst_hbm.at[pl.ds(off, CHUNK)], di_v.at[0, 0])
            pltpu.async_copy(y_hbm.at[si_v.at[0, 0]], rows_v.at[0], gsems[0])
            wait_gather(0)
            pltpu.sync_copy(rows_v.at[0], acc_sh.at[di_v.at[0, 0]], add=True)
        plsc.subcore_barrier()

        pltpu.sync_copy(acc_sh.at[pl.ds(r0, npt), :],
                        out_hbm.at[c, pl.ds(r0, npt), :])

    return k(src_h, dst_h, y, zeros_h)


def _tc_scale(xw, degp):
    n = xw.shape[0]

    def body(xw_ref, degp_ref, y_ref):
        deg = degp_ref[0, 0] + degp_ref[1, 0] + 1.0
        dinv = lax.rsqrt(deg)
        y_ref[:n] = xw_ref[...] * dinv[:n][:, None]
        y_ref[n:] = jnp.zeros((NPAD - n, D), jnp.float32)

    return pl.pallas_call(
        body,
        out_shape=jax.ShapeDtypeStruct((NPAD, D), jnp.float32),
    )(xw, degp)


def _tc_final(partials, y, degp, n):
    def body(p_ref, y_ref, degp_ref, out_ref):
        deg = degp_ref[0, 0] + degp_ref[1, 0] + 1.0
        dinv = lax.rsqrt(deg)
        acc = p_ref[0][:n] + p_ref[1][:n] + y_ref[:n]
        out_ref[...] = acc * dinv[:n][:, None]

    return pl.pallas_call(
        body,
        out_shape=jax.ShapeDtypeStruct((n, D), jnp.float32),
    )(partials, y, degp)


def kernel(x, edge_index, initial_weight, w_ih, w_hh, b_ih, b_hh):
    n = x.shape[0]
    e = edge_index.shape[1]
    assert e % (NW * HGS * CHUNK) == 0 and n < NPAD

    b_ih2 = b_ih.reshape(1, 3 * D)
    b_hh2 = b_hh.reshape(1, 3 * D)
    zeros_h = jnp.zeros((NPAD // NS, D), jnp.float32)

    src_h, dst_h = _tc_split(edge_index)
    degp = _sc_degree(dst_h)
    xw = _tc_gru_xw(x, initial_weight, w_ih, w_hh, b_ih2, b_hh2)
    y = _tc_scale(xw, degp)
    partials = _sc_scatter(src_h, dst_h, y, zeros_h)
    return _tc_final(partials, y, degp, n)


# speedup vs baseline: 1.3606x; 1.0191x over previous
import functools

import jax
import jax.numpy as jnp
from jax import lax
from jax.experimental import pallas as pl
from jax.experimental.pallas import tpu as pltpu
from jax.experimental.pallas import tpu_sc as plsc

D = 128
NC = 2
NS = 16
NW = NC * NS
CHUNK = 80
NB = 4
HGS = 5
NPAD = 10240


def _tc_gru_xw(x, w0, w_ih, w_hh, b_ih2, b_hh2):
    def body(x_ref, w0_ref, wih_ref, whh_ref, bih_ref, bhh_ref, out_ref):
        w = w0_ref[...]
        gi = lax.dot_general(w, wih_ref[...], (((1,), (1,)), ((), ())),
                             preferred_element_type=jnp.float32) + bih_ref[...]
        gh = lax.dot_general(w, whh_ref[...], (((1,), (1,)), ((), ())),
                             preferred_element_type=jnp.float32) + bhh_ref[...]
        r = jax.nn.sigmoid(gi[:, :D] + gh[:, :D])
        z = jax.nn.sigmoid(gi[:, D:2 * D] + gh[:, D:2 * D])
        n = jnp.tanh(gi[:, 2 * D:] + r * gh[:, 2 * D:])
        wt = (1.0 - z) * n + z * w
        out_ref[...] = jnp.dot(x_ref[...], wt,
                               preferred_element_type=jnp.float32)

    return pl.pallas_call(
        body,
        out_shape=jax.ShapeDtypeStruct((x.shape[0], D), jnp.float32),
    )(x, w0, w_ih, w_hh, b_ih2, b_hh2)


def _tc_split(edges):
    e = edges.shape[1]

    def body(e_ref, s_ref, d_ref):
        s_ref[...] = e_ref[0]
        d_ref[...] = e_ref[1]

    return pl.pallas_call(
        body,
        out_shape=(
            jax.ShapeDtypeStruct((e,), jnp.int32),
            jax.ShapeDtypeStruct((e,), jnp.int32),
        ),
    )(edges)


def _sc_degree(edges):
    e = edges.shape[1]
    hch = 128
    nch = e // hch
    cpt = nch // NW
    rem = nch - cpt * NW
    hgs = 13
    ngrp = cpt // hgs
    assert ngrp * hgs == cpt and e % hch == 0
    spt = NPAD // NS
    mesh = plsc.VectorSubcoreMesh(core_axis_name="c", subcore_axis_name="s")

    @functools.partial(
        pl.kernel,
        out_type=jax.ShapeDtypeStruct((NC, 1, NPAD), jnp.float32),
        mesh=mesh,
        scratch_types=[
            pltpu.VMEM((2, hgs, 2, hch), jnp.int32),
            pltpu.VMEM((hch,), jnp.float32),
            pltpu.VMEM((spt,), jnp.float32),
            pltpu.VMEM_SHARED((NPAD,), jnp.float32),
            pltpu.SemaphoreType.DMA,
            [pltpu.SemaphoreType.DMA] * 2,
        ],
    )
    def k(edge_hbm, out_hbm, idx_v, ones_v, buf_v, acc_sh, sem, stsems):
        c = lax.axis_index("c")
        s = lax.axis_index("s")
        wid = s * NC + c
        base = wid * cpt

        def fill_zero(i, _):
            buf_v[pl.ds(i * 16, 16)] = jnp.zeros((16,), jnp.float32)
            return 0
        lax.fori_loop(0, spt // 16, fill_zero, 0)

        def fill_one(i, _):
            ones_v[pl.ds(i * 16, 16)] = jnp.ones((16,), jnp.float32)
            return 0
        lax.fori_loop(0, hch // 16, fill_one, 0)

        def issue_stage(m, buf, sem2):
            off = (base + m * hgs) * hch
            for t in range(hgs):
                pltpu.async_copy(edge_hbm.at[:, pl.ds(off + t * hch, hch)],
                                 idx_v.at[buf, t], sem2)

        def wait_stage(buf, sem2):
            for t in range(hgs):
                pltpu.make_async_copy(edge_hbm.at[:, pl.ds(0, hch)],
                                      idx_v.at[buf, t], sem2).wait()

        pltpu.sync_copy(buf_v, acc_sh.at[pl.ds(s * spt, spt)])
        for t in range(hgs):
            pltpu.sync_copy(edge_hbm.at[:, pl.ds((base + t) * hch, hch)],
                            idx_v.at[0, t])
        plsc.subcore_barrier()

        def grp(g, _):
            p = lax.rem(g, 2)
            even = p == 0

            @pl.when(jnp.logical_and(g > 0, even))
            def _w0():
                wait_stage(p, stsems[0])

            @pl.when(jnp.logical_and(g > 0, jnp.logical_not(even)))
            def _w1():
                wait_stage(p, stsems[1])

            @pl.when(jnp.logical_and(g + 1 < ngrp, even))
            def _s1():
                issue_stage(g + 1, 1 - p, stsems[1])

            @pl.when(jnp.logical_and(g + 1 < ngrp, jnp.logical_not(even)))
            def _s0():
                issue_stage(g + 1, 1 - p, stsems[0])

            for t in range(hgs):
                pltpu.async_copy(ones_v, acc_sh.at[idx_v.at[p, t, 1]], sem,
                                 add=True)
            for t in range(hgs):
                pltpu.make_async_copy(ones_v, acc_sh.at[idx_v.at[p, t, 1]],
                                      sem).wait()
            return 0
        lax.fori_loop(0, ngrp, grp, 0)
        @pl.when(wid < rem)
        def _tail():
            off = (NW * cpt + wid) * hch
            pltpu.sync_copy(edge_hbm.at[:, pl.ds(off, hch)], idx_v.at[0, 0])
            pltpu.sync_copy(ones_v, acc_sh.at[idx_v.at[0, 0, 1]], add=True)
        plsc.subcore_barrier()

        pltpu.sync_copy(acc_sh.at[pl.ds(s * spt, spt)], buf_v)
        pltpu.sync_copy(buf_v, out_hbm.at[c, 0, pl.ds(s * spt, spt)])

    return k(edges)


def _sc_scatter(src_h, dst_h, y, zeros_h):
    e = src_h.shape[0]
    ept = e // NW
    nch = ept // CHUNK
    ngrp = nch // NB
    tail = nch - ngrp * NB
    npt = NPAD // NS
    mesh = plsc.VectorSubcoreMesh(core_axis_name="c", subcore_axis_name="s")

    @functools.partial(
        pl.kernel,
        out_type=jax.ShapeDtypeStruct((NC, NPAD, D), jnp.float32),
        mesh=mesh,
        scratch_types=[
            pltpu.VMEM((3, NB, CHUNK), jnp.int32),
            pltpu.VMEM((3, NB, CHUNK), jnp.int32),
            pltpu.VMEM((NB, CHUNK, D), jnp.float32),
            pltpu.VMEM_SHARED((NPAD, D), jnp.float32),
            [pltpu.SemaphoreType.DMA] * NB,
            [pltpu.SemaphoreType.DMA] * NB,
            [pltpu.SemaphoreType.DMA] * 2,
        ],
    )
    def k(src_hbm, dst_hbm, y_hbm, z_hbm, out_hbm,
          si_v, di_v, rows_v, acc_sh, gsems, ssems, stsems):
        c = lax.axis_index("c")
        s = lax.axis_index("s")
        wid = s * NC + c
        base = wid * ept
        r0 = s * npt

        def issue_stage(m, sem):
            b = lax.rem(m, 3)
            off = base + m * (NB * CHUNK)
            for t in range(NB):
                pltpu.async_copy(
                    src_hbm.at[pl.ds(off + t * CHUNK, CHUNK)],
                    si_v.at[b, t], sem)
            for t in range(NB):
                pltpu.async_copy(
                    dst_hbm.at[pl.ds(off + t * CHUNK, CHUNK)],
                    di_v.at[b, t], sem)

        def wait_stage(bg, sem):
            for t in range(NB):
                pltpu.make_async_copy(src_hbm.at[pl.ds(0, CHUNK)],
                                      si_v.at[bg, t], sem).wait()
            for t in range(NB):
                pltpu.make_async_copy(dst_hbm.at[pl.ds(0, CHUNK)],
                                      di_v.at[bg, t], sem).wait()

        def wait_gather(b):
            pltpu.make_async_copy(
                y_hbm.at[si_v.at[0, 0]], rows_v.at[b], gsems[b]).wait()

        def wait_scatter(b):
            pltpu.make_async_copy(
                rows_v.at[b], acc_sh.at[di_v.at[0, 0]], ssems[b]).wait()

        pltpu.sync_copy(z_hbm, acc_sh.at[pl.ds(r0, npt), :])
        for t in range(NB):
            pltpu.sync_copy(src_hbm.at[pl.ds(base + t * CHUNK, CHUNK)],
                            si_v.at[0, t])
            pltpu.sync_copy(dst_hbm.at[pl.ds(base + t * CHUNK, CHUNK)],
                            di_v.at[0, t])
        issue_stage(1, stsems[1])
        issue_stage(2, stsems[0])
        pltpu.async_copy(y_hbm.at[si_v.at[0, 0]], rows_v.at[0], gsems[0])
        pltpu.async_copy(y_hbm.at[si_v.at[0, 1]], rows_v.at[1], gsems[1])
        plsc.subcore_barrier()

        def grp(g, _):
            bg = lax.rem(g, 3)
            bg1 = lax.rem(g + 1, 3)
            even = lax.rem(g, 2) == 0

            @pl.when(jnp.logical_and(g > 0, even))
            def _ws0():
                wait_stage(bg, stsems[0])

            @pl.when(jnp.logical_and(g > 0, jnp.logical_not(even)))
            def _ws1():
                wait_stage(bg, stsems[1])

            for t in range(NB):
                b = t % NB
                jb = (t + 2) % NB
                if t < 2:
                    @pl.when(g > 0)
                    def _wsct():
                        wait_scatter(jb)
                else:
                    wait_scatter(jb)
                if t == 2:
                    @pl.when(jnp.logical_and(g + 2 < ngrp, even))
                    def _st0():
                        issue_stage(g + 2, stsems[0])

                    @pl.when(jnp.logical_and(g + 2 < ngrp,
                                             jnp.logical_not(even)))
                    def _st1():
                        issue_stage(g + 2, stsems[1])
                if t < NB - 2:
                    nxt = si_v.at[bg, t + 2]
                else:
                    nxt = si_v.at[bg1, t + 2 - NB]
                pltpu.async_copy(y_hbm.at[nxt], rows_v.at[jb], gsems[jb])
                wait_gather(b)
                pltpu.async_copy(rows_v.at[b], acc_sh.at[di_v.at[bg, t]],
                                 ssems[b], add=True)
            return 0
        lax.fori_loop(0, ngrp, grp, 0)
        wait_scatter(NB - 2)
        wait_scatter(NB - 1)
        wait_gather(0)
        wait_gather(1)
        for u in range(tail):
            off = base + (ngrp * NB + u) * CHUNK
            pltpu.sync_copy(src_hbm.at[pl.ds(off, CHUNK)], si_v.at[0, 0])
            pltpu.sync_copy(dst_hbm.at[pl.ds(off, CHUNK)], di_v.at[0, 0])
            pltpu.async_copy(y_hbm.at[si_v.at[0, 0]], rows_v.at[0], gsems[0])
            wait_gather(0)
            pltpu.sync_copy(rows_v.at[0], acc_sh.at[di_v.at[0, 0]], add=True)
        plsc.subcore_barrier()

        pltpu.sync_copy(acc_sh.at[pl.ds(r0, npt), :],
                        out_hbm.at[c, pl.ds(r0, npt), :])

    return k(src_h, dst_h, y, zeros_h)


def _tc_scale(xw, degp):
    n = xw.shape[0]

    def body(xw_ref, degp_ref, y_ref):
        deg = degp_ref[0, 0] + degp_ref[1, 0] + 1.0
        dinv = lax.rsqrt(deg)
        y_ref[:n] = xw_ref[...] * dinv[:n][:, None]
        y_ref[n:] = jnp.zeros((NPAD - n, D), jnp.float32)

    return pl.pallas_call(
        body,
        out_shape=jax.ShapeDtypeStruct((NPAD, D), jnp.float32),
    )(xw, degp)


def _tc_final(partials, y, degp, n):
    def body(p_ref, y_ref, degp_ref, out_ref):
        deg = degp_ref[0, 0] + degp_ref[1, 0] + 1.0
        dinv = lax.rsqrt(deg)
        acc = p_ref[0][:n] + p_ref[1][:n] + y_ref[:n]
        out_ref[...] = acc * dinv[:n][:, None]

    return pl.pallas_call(
        body,
        out_shape=jax.ShapeDtypeStruct((n, D), jnp.float32),
    )(partials, y, degp)


def kernel(x, edge_index, initial_weight, w_ih, w_hh, b_ih, b_hh):
    n = x.shape[0]
    e = edge_index.shape[1]
    assert e % (NW * HGS * CHUNK) == 0 and n < NPAD

    b_ih2 = b_ih.reshape(1, 3 * D)
    b_hh2 = b_hh.reshape(1, 3 * D)
    zeros_h = jnp.zeros((NPAD // NS, D), jnp.float32)

    degp = _sc_degree(edge_index)
    src_h, dst_h = _tc_split(edge_index)
    xw = _tc_gru_xw(x, initial_weight, w_ih, w_hh, b_ih2, b_hh2)
    y = _tc_scale(xw, degp)
    partials = _sc_scatter(src_h, dst_h, y, zeros_h)
    return _tc_final(partials, y, degp, n)


# speedup vs baseline: 1.3637x; 1.0023x over previous
import functools

import jax
import jax.numpy as jnp
from jax import lax
from jax.experimental import pallas as pl
from jax.experimental.pallas import tpu as pltpu
from jax.experimental.pallas import tpu_sc as plsc

D = 128
NC = 2
NS = 16
NW = NC * NS
CHUNK = 80
NB = 4
NPAD = 10240


def _tc_gru_xw(x, w0, w_ih, w_hh, b_ih2, b_hh2):
    def body(x_ref, w0_ref, wih_ref, whh_ref, bih_ref, bhh_ref, out_ref):
        w = w0_ref[...]
        gi = lax.dot_general(w, wih_ref[...], (((1,), (1,)), ((), ())),
                             preferred_element_type=jnp.float32) + bih_ref[...]
        gh = lax.dot_general(w, whh_ref[...], (((1,), (1,)), ((), ())),
                             preferred_element_type=jnp.float32) + bhh_ref[...]
        r = jax.nn.sigmoid(gi[:, :D] + gh[:, :D])
        z = jax.nn.sigmoid(gi[:, D:2 * D] + gh[:, D:2 * D])
        n = jnp.tanh(gi[:, 2 * D:] + r * gh[:, 2 * D:])
        wt = (1.0 - z) * n + z * w
        out_ref[...] = jnp.dot(x_ref[...], wt,
                               preferred_element_type=jnp.float32)

    return pl.pallas_call(
        body,
        out_shape=jax.ShapeDtypeStruct((x.shape[0], D), jnp.float32),
    )(x, w0, w_ih, w_hh, b_ih2, b_hh2)


def _tc_split(edges):
    e = edges.shape[1]

    def body(e_ref, s_ref, d_ref):
        s_ref[...] = e_ref[0]
        d_ref[...] = e_ref[1]

    return pl.pallas_call(
        body,
        out_shape=(
            jax.ShapeDtypeStruct((e,), jnp.int32),
            jax.ShapeDtypeStruct((e,), jnp.int32),
        ),
    )(edges)


def _sc_degree(edges):
    e = edges.shape[1]
    hch = 128
    nch = e // hch
    cpt = nch // NW
    rem = nch - cpt * NW
    hgs = 13
    ngrp = cpt // hgs
    assert ngrp * hgs == cpt and e % hch == 0
    spt = NPAD // NS
    mesh = plsc.VectorSubcoreMesh(core_axis_name="c", subcore_axis_name="s")

    @functools.partial(
        pl.kernel,
        out_type=jax.ShapeDtypeStruct((NC, 1, NPAD), jnp.float32),
        mesh=mesh,
        scratch_types=[
            pltpu.VMEM((2, hgs, 2, hch), jnp.int32),
            pltpu.VMEM((hch,), jnp.float32),
            pltpu.VMEM((spt,), jnp.float32),
            pltpu.VMEM_SHARED((NPAD,), jnp.float32),
            pltpu.SemaphoreType.DMA,
            [pltpu.SemaphoreType.DMA] * 2,
        ],
    )
    def k(edge_hbm, out_hbm, idx_v, ones_v, buf_v, acc_sh, sem, stsems):
        c = lax.axis_index("c")
        s = lax.axis_index("s")
        wid = s * NC + c
        base = wid * cpt

        def fill_zero(i, _):
            buf_v[pl.ds(i * 16, 16)] = jnp.zeros((16,), jnp.float32)
            return 0
        lax.fori_loop(0, spt // 16, fill_zero, 0)

        def fill_one(i, _):
            ones_v[pl.ds(i * 16, 16)] = jnp.ones((16,), jnp.float32)
            return 0
        lax.fori_loop(0, hch // 16, fill_one, 0)

        def issue_stage(m, buf, sem2):
            off = (base + m * hgs) * hch
            for t in range(hgs):
                pltpu.async_copy(edge_hbm.at[:, pl.ds(off + t * hch, hch)],
                                 idx_v.at[buf, t], sem2)

        def wait_stage(buf, sem2):
            for t in range(hgs):
                pltpu.make_async_copy(edge_hbm.at[:, pl.ds(0, hch)],
                                      idx_v.at[buf, t], sem2).wait()

        pltpu.sync_copy(buf_v, acc_sh.at[pl.ds(s * spt, spt)])
        for t in range(hgs):
            pltpu.sync_copy(edge_hbm.at[:, pl.ds((base + t) * hch, hch)],
                            idx_v.at[0, t])
        plsc.subcore_barrier()

        def grp(g, _):
            p = lax.rem(g, 2)
            even = p == 0

            @pl.when(jnp.logical_and(g > 0, even))
            def _w0():
                wait_stage(p, stsems[0])

            @pl.when(jnp.logical_and(g > 0, jnp.logical_not(even)))
            def _w1():
                wait_stage(p, stsems[1])

            @pl.when(jnp.logical_and(g + 1 < ngrp, even))
            def _s1():
                issue_stage(g + 1, 1 - p, stsems[1])

            @pl.when(jnp.logical_and(g + 1 < ngrp, jnp.logical_not(even)))
            def _s0():
                issue_stage(g + 1, 1 - p, stsems[0])

            for t in range(hgs):
                pltpu.async_copy(ones_v, acc_sh.at[idx_v.at[p, t, 1]], sem,
                                 add=True)
            for t in range(hgs):
                pltpu.make_async_copy(ones_v, acc_sh.at[idx_v.at[p, t, 1]],
                                      sem).wait()
            return 0
        lax.fori_loop(0, ngrp, grp, 0)
        @pl.when(wid < rem)
        def _tail():
            off = (NW * cpt + wid) * hch
            pltpu.sync_copy(edge_hbm.at[:, pl.ds(off, hch)], idx_v.at[0, 0])
            pltpu.sync_copy(ones_v, acc_sh.at[idx_v.at[0, 0, 1]], add=True)
        plsc.subcore_barrier()

        pltpu.sync_copy(acc_sh.at[pl.ds(s * spt, spt)], buf_v)
        pltpu.sync_copy(buf_v, out_hbm.at[c, 0, pl.ds(s * spt, spt)])

    return k(edges)


def _sc_scatter(src_h, dst_h, y, zeros_h):
    e = src_h.shape[0]
    ept = e // NW
    nch = ept // CHUNK
    ngrp = nch // NB
    tail = nch - ngrp * NB
    npt = NPAD // NS
    mesh = plsc.VectorSubcoreMesh(core_axis_name="c", subcore_axis_name="s")

    @functools.partial(
        pl.kernel,
        out_type=jax.ShapeDtypeStruct((NC, NPAD, D), jnp.float32),
        mesh=mesh,
        scratch_types=[
            pltpu.VMEM((3, NB, CHUNK), jnp.int32),
            pltpu.VMEM((3, NB, CHUNK), jnp.int32),
            pltpu.VMEM((NB, CHUNK, D), jnp.float32),
            pltpu.VMEM_SHARED((NPAD, D), jnp.float32),
            [pltpu.SemaphoreType.DMA] * NB,
            [pltpu.SemaphoreType.DMA] * NB,
            [pltpu.SemaphoreType.DMA] * 2,
        ],
    )
    def k(src_hbm, dst_hbm, y_hbm, z_hbm, out_hbm,
          si_v, di_v, rows_v, acc_sh, gsems, ssems, stsems):
        c = lax.axis_index("c")
        s = lax.axis_index("s")
        wid = s * NC + c
        base = wid * ept
        r0 = s * npt

        def issue_stage(m, sem):
            b = lax.rem(m, 3)
            off = base + m * (NB * CHUNK)
            for t in range(NB):
                pltpu.async_copy(
                    src_hbm.at[pl.ds(off + t * CHUNK, CHUNK)],
                    si_v.at[b, t], sem)
            for t in range(NB):
                pltpu.async_copy(
                    dst_hbm.at[pl.ds(off + t * CHUNK, CHUNK)],
                    di_v.at[b, t], sem)

        def wait_stage(bg, sem):
            for t in range(NB):
                pltpu.make_async_copy(src_hbm.at[pl.ds(0, CHUNK)],
                                      si_v.at[bg, t], sem).wait()
            for t in range(NB):
                pltpu.make_async_copy(dst_hbm.at[pl.ds(0, CHUNK)],
                                      di_v.at[bg, t], sem).wait()

        def wait_gather(b):
            pltpu.make_async_copy(
                y_hbm.at[si_v.at[0, 0]], rows_v.at[b], gsems[b]).wait()

        def wait_scatter(b):
            pltpu.make_async_copy(
                rows_v.at[b], acc_sh.at[di_v.at[0, 0]], ssems[b]).wait()

        pltpu.sync_copy(z_hbm, acc_sh.at[pl.ds(r0, npt), :])
        for t in range(NB):
            pltpu.sync_copy(src_hbm.at[pl.ds(base + t * CHUNK, CHUNK)],
                            si_v.at[0, t])
            pltpu.sync_copy(dst_hbm.at[pl.ds(base + t * CHUNK, CHUNK)],
                            di_v.at[0, t])
        issue_stage(1, stsems[1])
        issue_stage(2, stsems[0])
        pltpu.async_copy(y_hbm.at[si_v.at[0, 0]], rows_v.at[0], gsems[0])
        pltpu.async_copy(y_hbm.at[si_v.at[0, 1]], rows_v.at[1], gsems[1])
        plsc.subcore_barrier()

        def grp(g, _):
            bg = lax.rem(g, 3)
            bg1 = lax.rem(g + 1, 3)
            even = lax.rem(g, 2) == 0

            @pl.when(jnp.logical_and(g > 0, even))
            def _ws0():
                wait_stage(bg, stsems[0])

            @pl.when(jnp.logical_and(g > 0, jnp.logical_not(even)))
            def _ws1():
                wait_stage(bg, stsems[1])

            for t in range(NB):
                b = t % NB
                jb = (t + 2) % NB
                if t < 2:
                    @pl.when(g > 0)
                    def _wsct():
                        wait_scatter(jb)
                else:
                    wait_scatter(jb)
                if t == 2:
                    @pl.when(jnp.logical_and(g + 2 < ngrp, even))
                    def _st0():
                        issue_stage(g + 2, stsems[0])

                    @pl.when(jnp.logical_and(g + 2 < ngrp,
                                             jnp.logical_not(even)))
                    def _st1():
                        issue_stage(g + 2, stsems[1])
                if t < NB - 2:
                    nxt = si_v.at[bg, t + 2]
                else:
                    nxt = si_v.at[bg1, t + 2 - NB]
                pltpu.async_copy(y_hbm.at[nxt], rows_v.at[jb], gsems[jb])
                wait_gather(b)
                pltpu.async_copy(rows_v.at[b], acc_sh.at[di_v.at[bg, t]],
                                 ssems[b], add=True)
            return 0
        lax.fori_loop(0, ngrp, grp, 0)
        wait_scatter(NB - 2)
        wait_scatter(NB - 1)
        wait_gather(0)
        wait_gather(1)
        for u in range(tail):
            off = base + (ngrp * NB + u) * CHUNK
            pltpu.sync_copy(src_hbm.at[pl.ds(off, CHUNK)], si_v.at[0, 0])
            pltpu.sync_copy(dst_hbm.at[pl.ds(off, CHUNK)], di_v.at[0, 0])
            pltpu.async_copy(y_hbm.at[si_v.at[0, 0]], rows_v.at[0], gsems[0])
            wait_gather(0)
            pltpu.sync_copy(rows_v.at[0], acc_sh.at[di_v.at[0, 0]], add=True)
        plsc.subcore_barrier()

        pltpu.sync_copy(acc_sh.at[pl.ds(r0, npt), :],
                        out_hbm.at[c, pl.ds(r0, npt), :])

    return k(src_h, dst_h, y, zeros_h)


def _tc_scale(xw, degp):
    n = xw.shape[0]

    def body(xw_ref, degp_ref, y_ref):
        deg = degp_ref[0, 0] + degp_ref[1, 0] + 1.0
        dinv = lax.rsqrt(deg)
        y_ref[:n] = xw_ref[...] * dinv[:n][:, None]
        y_ref[n:] = jnp.zeros((NPAD - n, D), jnp.float32)

    return pl.pallas_call(
        body,
        out_shape=jax.ShapeDtypeStruct((NPAD, D), jnp.float32),
    )(xw, degp)


def _tc_final(partials, y, degp, n):
    def body(p_ref, y_ref, degp_ref, out_ref):
        deg = degp_ref[0, 0] + degp_ref[1, 0] + 1.0
        dinv = lax.rsqrt(deg)
        acc = p_ref[0][:n] + p_ref[1][:n] + y_ref[:n]
        out_ref[...] = acc * dinv[:n][:, None]

    return pl.pallas_call(
        body,
        out_shape=jax.ShapeDtypeStruct((n, D), jnp.float32),
    )(partials, y, degp)


def kernel(x, edge_index, initial_weight, w_ih, w_hh, b_ih, b_hh):
    n = x.shape[0]
    e = edge_index.shape[1]
    assert e % (NW * CHUNK) == 0 and n < NPAD

    b_ih2 = b_ih.reshape(1, 3 * D)
    b_hh2 = b_hh.reshape(1, 3 * D)
    zeros_h = jnp.zeros((NPAD // NS, D), jnp.float32)

    degp = _sc_degree(edge_index)
    src_h, dst_h = _tc_split(edge_index)
    xw = _tc_gru_xw(x, initial_weight, w_ih, w_hh, b_ih2, b_hh2)
    y = _tc_scale(xw, degp)
    partials = _sc_scatter(src_h, dst_h, y, zeros_h)
    return _tc_final(partials, y, degp, n)
